# baseline (speedup 1.0000x reference)
import jax
import jax.numpy as jnp
from jax.experimental import pallas as pl
from jax.experimental.pallas import tpu as pltpu

_B, _C, _HW = 64, 768, 576
_BBLK = 8


def _router_body(x_ref, w1_ref, b1_ref, w2_ref, b2_ref, out_ref, pooled_acc):
    i = pl.program_id(0)
    n = pl.num_programs(0)
    pooled_acc[pl.ds(i * _BBLK, _BBLK), :] = jnp.sum(x_ref[...], axis=2)

    @pl.when(i == n - 1)
    def _finish():
        pooled = pooled_acc[...] * (1.0 / _HW)
        h = pooled @ w1_ref[...] + b1_ref[...]
        h = 0.5 * h * (1.0 + jax.lax.erf(h * (2.0 ** -0.5)))
        logits = h @ w2_ref[...] + b2_ref[...]
        m = jnp.max(logits, axis=-1, keepdims=True)
        e = jnp.exp(logits - m)
        out_ref[...] = e / jnp.sum(e, axis=-1, keepdims=True)


def kernel(x, W1, b1, W2, b2):
    B, C, H, W = x.shape
    hw = H * W
    x3 = x.reshape(B, C, hw)
    num_blocks = B // _BBLK
    grid = (num_blocks,)
    return pl.pallas_call(
        _router_body,
        grid=grid,
        in_specs=[
            pl.BlockSpec((_BBLK, C, hw), lambda i: (i, 0, 0)),
            pl.BlockSpec((C, W1.shape[1]), lambda i: (0, 0)),
            pl.BlockSpec((W1.shape[1],), lambda i: (0,)),
            pl.BlockSpec((W1.shape[1], W2.shape[1]), lambda i: (0, 0)),
            pl.BlockSpec((W2.shape[1],), lambda i: (0,)),
        ],
        out_specs=pl.BlockSpec((B, W2.shape[1]), lambda i: (0, 0)),
        out_shape=jax.ShapeDtypeStruct((B, W2.shape[1]), jnp.float32),
        scratch_shapes=[pltpu.VMEM((B, C), jnp.float32)],
    )(x3, W1, b1, W2, b2)


# speedup vs baseline: 3.3167x; 3.3167x over previous
import jax
import jax.numpy as jnp
from jax.experimental import pallas as pl
from jax.experimental.pallas import tpu as pltpu

_BBLK = 8
_HWBLK = 144


def _router_body(x_ref, w1_ref, b1_ref, w2_ref, b2_ref, out_ref, pooled_acc):
    i = pl.program_id(0)
    j = pl.program_id(1)
    ni = pl.num_programs(0)
    nj = pl.num_programs(1)
    part = jnp.sum(x_ref[...], axis=1)

    @pl.when(j == 0)
    def _init():
        pooled_acc[pl.ds(i * _BBLK, _BBLK), :] = part

    @pl.when(j > 0)
    def _accum():
        pooled_acc[pl.ds(i * _BBLK, _BBLK), :] += part

    @pl.when((i == ni - 1) & (j == nj - 1))
    def _finish():
        pooled = pooled_acc[...] * (1.0 / (nj * _HWBLK))
        h = pooled @ w1_ref[...] + b1_ref[...]
        h = 0.5 * h * (1.0 + jax.lax.erf(h * (2.0 ** -0.5)))
        logits = h @ w2_ref[...] + b2_ref[...]
        m = jnp.max(logits, axis=-1, keepdims=True)
        e = jnp.exp(logits - m)
        out_ref[...] = e / jnp.sum(e, axis=-1, keepdims=True)


def kernel(x, W1, b1, W2, b2):
    B, C, H, W = x.shape
    hw = H * W
    xt = jnp.transpose(x, (0, 2, 3, 1)).reshape(B, hw, C)
    grid = (B // _BBLK, hw // _HWBLK)
    return pl.pallas_call(
        _router_body,
        grid=grid,
        in_specs=[
            pl.BlockSpec((_BBLK, _HWBLK, C), lambda i, j: (i, j, 0)),
            pl.BlockSpec((C, W1.shape[1]), lambda i, j: (0, 0)),
            pl.BlockSpec((W1.shape[1],), lambda i, j: (0,)),
            pl.BlockSpec((W1.shape[1], W2.shape[1]), lambda i, j: (0, 0)),
            pl.BlockSpec((W2.shape[1],), lambda i, j: (0,)),
        ],
        out_specs=pl.BlockSpec((B, W2.shape[1]), lambda i, j: (0, 0)),
        out_shape=jax.ShapeDtypeStruct((B, W2.shape[1]), jnp.float32),
        scratch_shapes=[pltpu.VMEM((B, C), jnp.float32)],
    )(xt, W1, b1, W2, b2)
